# baseline (speedup 1.0000x reference)
import jax
import jax.numpy as jnp
from jax.experimental import pallas as pl
from jax.experimental.pallas import tpu as pltpu

TOK = 16384
DM = 2048
NE = 64
CT = 512
DEPTH = 6
NCH = TOK // CT


def _gate_kernel(x_hbm, wt_ref, b_ref, o_hbm, xbuf, obuf, in_sems, out_sems):
    wb = wt_ref[...].astype(jnp.bfloat16)
    bias = b_ref[...]

    def in_copy(j):
        return pltpu.make_async_copy(
            x_hbm.at[pl.ds(j * CT, CT), :], xbuf.at[j % DEPTH], in_sems.at[j % DEPTH])

    def out_copy(j):
        return pltpu.make_async_copy(
            obuf.at[j % DEPTH], o_hbm.at[pl.ds(j * CT, CT), :], out_sems.at[j % DEPTH])

    for j in range(DEPTH - 1):
        in_copy(j).start()
    for i in range(NCH):
        s = i % DEPTH
        if i + DEPTH - 1 < NCH:
            in_copy(i + DEPTH - 1).start()
        in_copy(i).wait()
        if i >= DEPTH:
            out_copy(i - DEPTH).wait()
        xb = xbuf[s].astype(jnp.bfloat16)
        logits = jnp.dot(xb, wb, preferred_element_type=jnp.float32) + bias
        m = jnp.max(logits, axis=-1, keepdims=True)
        e = jnp.exp(logits - m)
        obuf[s] = e / jnp.sum(e, axis=-1, keepdims=True)
        out_copy(i).start()
    for i in range(max(NCH - DEPTH, 0), NCH):
        out_copy(i).wait()


def kernel(x, W, b):
    return pl.pallas_call(
        _gate_kernel,
        in_specs=[
            pl.BlockSpec(memory_space=pltpu.MemorySpace.HBM),
            pl.BlockSpec(memory_space=pltpu.MemorySpace.VMEM),
            pl.BlockSpec(memory_space=pltpu.MemorySpace.VMEM),
        ],
        out_specs=pl.BlockSpec(memory_space=pltpu.MemorySpace.HBM),
        out_shape=jax.ShapeDtypeStruct((TOK, NE), jnp.float32),
        scratch_shapes=[
            pltpu.VMEM((DEPTH, CT, DM), jnp.float32),
            pltpu.VMEM((DEPTH, CT, NE), jnp.float32),
            pltpu.SemaphoreType.DMA((DEPTH,)),
            pltpu.SemaphoreType.DMA((DEPTH,)),
        ],
    )(x, W.T, b.reshape(1, NE))


# speedup vs baseline: 1.0473x; 1.0473x over previous
import jax
import jax.numpy as jnp
from jax.experimental import pallas as pl
from jax.experimental.pallas import tpu as pltpu

TOK = 16384
DM = 2048
NE = 64
CT = 512
DEPTH = 8
NCH = TOK // CT


def _gate_kernel(x_hbm, wt_ref, b_ref, o_hbm, xbuf, obuf, in_sems, out_sems):
    wb = wt_ref[...].astype(jnp.bfloat16)
    bias = b_ref[...]

    def in_copy(j, slot):
        return pltpu.make_async_copy(
            x_hbm.at[pl.ds(j * CT, CT), :], xbuf.at[slot], in_sems.at[slot])

    def out_copy(j, slot):
        return pltpu.make_async_copy(
            obuf.at[slot], o_hbm.at[pl.ds(j * CT, CT), :], out_sems.at[slot])

    for j in range(DEPTH - 1):
        in_copy(j, j).start()

    def body(i, carry):
        s = jnp.bitwise_and(i, DEPTH - 1)
        ps = jnp.bitwise_and(i + DEPTH - 1, DEPTH - 1)

        @pl.when(i < NCH - DEPTH + 1)
        def _():
            in_copy(i + DEPTH - 1, ps).start()

        in_copy(i, s).wait()

        @pl.when(i >= DEPTH)
        def _():
            out_copy(i - DEPTH, s).wait()

        xb = xbuf[s].astype(jnp.bfloat16)
        logits = jnp.dot(xb, wb, preferred_element_type=jnp.float32) + bias
        m = jnp.max(logits, axis=-1, keepdims=True)
        e = jnp.exp(logits - m)
        obuf[s] = e / jnp.sum(e, axis=-1, keepdims=True)
        out_copy(i, s).start()
        return carry

    jax.lax.fori_loop(0, NCH, body, 0)
    for k in range(DEPTH):
        j = NCH - DEPTH + k
        out_copy(j, j % DEPTH).wait()


def kernel(x, W, b):
    return pl.pallas_call(
        _gate_kernel,
        in_specs=[
            pl.BlockSpec(memory_space=pltpu.MemorySpace.HBM),
            pl.BlockSpec(memory_space=pltpu.MemorySpace.VMEM),
            pl.BlockSpec(memory_space=pltpu.MemorySpace.VMEM),
        ],
        out_specs=pl.BlockSpec(memory_space=pltpu.MemorySpace.HBM),
        out_shape=jax.ShapeDtypeStruct((TOK, NE), jnp.float32),
        scratch_shapes=[
            pltpu.VMEM((DEPTH, CT, DM), jnp.float32),
            pltpu.VMEM((DEPTH, CT, NE), jnp.float32),
            pltpu.SemaphoreType.DMA((DEPTH,)),
            pltpu.SemaphoreType.DMA((DEPTH,)),
        ],
    )(x, W.T, b.reshape(1, NE))
